# Initial kernel scaffold; baseline (speedup 1.0000x reference)
import jax
import jax.numpy as jnp
from jax.experimental import pallas as pl


def kernel(img):
    raise NotImplementedError("write your pallas kernel here")


# speedup vs baseline: 1.5036x; 1.5036x over previous
import jax
import jax.numpy as jnp
import numpy as np
from jax import lax
from jax.experimental import pallas as pl
from jax.experimental.pallas import tpu as pltpu

_C, _H, _W = 192, 384, 384
_T = 64
_NH, _NW = _H // _T, _W // _T
_NT = _NH * _NW
_NB = 18
_CC = 64
_NC = _C // _CC

_PERM = np.array([
    35, 31, 7, 4, 29, 16, 19, 2, 34, 5, 30, 3, 22, 6, 18, 10, 11, 32,
    15, 20, 8, 24, 9, 25, 13, 33, 14, 17, 23, 0, 21, 26, 1, 28, 27, 12,
], dtype=np.int32)
_Q = _PERM % _NB
_DESTS = np.concatenate([np.where(_Q == r)[0] for r in range(_NB)]).astype(np.int32)
assert _DESTS.shape == (2 * _NB,)


def _score_topk_body(in_ref, top_ref, scores_ref):
    th = pl.program_id(0)
    c = pl.program_id(1)
    x = in_ref[...].reshape(_CC * _T, _W)
    a = jnp.abs(x)
    s = jnp.sum(a, axis=0, keepdims=True)
    lanes = lax.broadcasted_iota(jnp.int32, (1, _W), 1)
    for j in range(_NW):
        pj = jnp.sum(jnp.where((lanes >= j * _T) & (lanes < (j + 1) * _T), s, 0.0))
        idx = th * _NW + j

        @pl.when(c == 0)
        def _(pj=pj, idx=idx):
            scores_ref[idx] = pj

        @pl.when(c > 0)
        def _(pj=pj, idx=idx):
            scores_ref[idx] = scores_ref[idx] + pj

    @pl.when((th == _NH - 1) & (c == _NC - 1))
    def _():
        def outer(i, _):
            si = scores_ref[i]

            def inner(j, r):
                sj = scores_ref[j]
                gt = (sj > si) | ((sj == si) & (j < i))
                return r + gt.astype(jnp.int32)

            r = lax.fori_loop(0, _NT, inner, 0)

            @pl.when(r < _NB)
            def _():
                top_ref[r] = i

            return 0

        lax.fori_loop(0, _NT, outer, 0)


def _copy_body(idx_ref, dest_ref, img_ref, out_ref, bufs, in_sem, out_sem):
    r = pl.program_id(0)

    def start_in(rr, slot):
        t = idx_ref[rr]
        sh = (t // _NW) * _T
        sw = t % _NW
        src = img_ref.at[:, pl.ds(sh, _T), sw, :]
        pltpu.make_async_copy(src, bufs.at[slot], in_sem.at[slot]).start()

    slot = r % 2

    @pl.when(r == 0)
    def _():
        start_in(r, slot)

    @pl.when(r > 0)
    def _():
        other = (r + 1) % 2
        for e in range(2):
            pltpu.make_async_copy(bufs.at[other], out_ref.at[:, pl.ds(0, _T), 0, :], out_sem.at[other, e]).wait()

    @pl.when(r < _NB - 1)
    def _():
        start_in(r + 1, (r + 1) % 2)

    pltpu.make_async_copy(img_ref.at[:, pl.ds(0, _T), 0, :], bufs.at[slot], in_sem.at[slot]).wait()

    for e in range(2):
        k = dest_ref[2 * r + e]
        oh = (k // _NW) * _T
        ow = k % _NW
        dst = out_ref.at[:, pl.ds(oh, _T), ow, :]
        pltpu.make_async_copy(bufs.at[slot], dst, out_sem.at[slot, e]).start()

    @pl.when(r == _NB - 1)
    def _():
        for e in range(2):
            pltpu.make_async_copy(bufs.at[slot], out_ref.at[:, pl.ds(0, _T), 0, :], out_sem.at[slot, e]).wait()


def kernel(img):
    img4 = img.reshape(_C, _NH, _T, _W)

    top_idx = pl.pallas_call(
        _score_topk_body,
        grid=(_NH, _NC),
        in_specs=[
            pl.BlockSpec((_CC, 1, _T, _W), lambda th, c: (c, th, 0, 0)),
        ],
        out_specs=pl.BlockSpec(memory_space=pltpu.SMEM),
        out_shape=jax.ShapeDtypeStruct((_NB,), jnp.int32),
        scratch_shapes=[pltpu.SMEM((_NT,), jnp.float32)],
        compiler_params=pltpu.CompilerParams(
            dimension_semantics=("arbitrary", "arbitrary"),
        ),
    )(img4)

    dests = jnp.asarray(_DESTS)
    imgc = img.reshape(_C, _H, _NW, _T)

    out = pl.pallas_call(
        _copy_body,
        grid=(_NB,),
        in_specs=[
            pl.BlockSpec(memory_space=pltpu.SMEM),
            pl.BlockSpec(memory_space=pltpu.SMEM),
            pl.BlockSpec(memory_space=pl.ANY),
        ],
        out_specs=pl.BlockSpec(memory_space=pl.ANY),
        out_shape=jax.ShapeDtypeStruct((_C, _H, _NW, _T), jnp.float32),
        scratch_shapes=[
            pltpu.VMEM((2, _C, _T, _T), jnp.float32),
            pltpu.SemaphoreType.DMA((2,)),
            pltpu.SemaphoreType.DMA((2, 2)),
        ],
        compiler_params=pltpu.CompilerParams(
            dimension_semantics=("arbitrary",),
        ),
    )(top_idx, dests, imgc)

    return out.reshape(_C, _H, _W)


# speedup vs baseline: 6.4438x; 4.2855x over previous
import jax
import jax.numpy as jnp
import numpy as np
from jax import lax
from jax.experimental import pallas as pl
from jax.experimental.pallas import tpu as pltpu

_C, _H, _W = 192, 384, 384
_T = 64
_NH, _NW = _H // _T, _W // _T
_NT = _NH * _NW
_NB = 18
_CC = 64
_NC = _C // _CC

_PERM = np.array([
    35, 31, 7, 4, 29, 16, 19, 2, 34, 5, 30, 3, 22, 6, 18, 10, 11, 32,
    15, 20, 8, 24, 9, 25, 13, 33, 14, 17, 23, 0, 21, 26, 1, 28, 27, 12,
], dtype=np.int32)
_Q = _PERM % _NB
_DESTS = np.concatenate([np.where(_Q == r)[0] for r in range(_NB)]).astype(np.int32)
assert _DESTS.shape == (2 * _NB,)


def _score_topk_body(in_ref, top_ref, scores_ref):
    th = pl.program_id(0)
    c = pl.program_id(1)
    a = jnp.abs(in_ref[...])
    s = jnp.sum(a, axis=(0, 1)).reshape(1, _W)
    lanes = lax.broadcasted_iota(jnp.int32, (1, _W), 1)
    for j in range(_NW):
        pj = jnp.sum(jnp.where((lanes >= j * _T) & (lanes < (j + 1) * _T), s, 0.0))
        idx = th * _NW + j

        @pl.when(c == 0)
        def _(pj=pj, idx=idx):
            scores_ref[idx] = pj

        @pl.when(c > 0)
        def _(pj=pj, idx=idx):
            scores_ref[idx] = scores_ref[idx] + pj

    @pl.when((th == _NH - 1) & (c == _NC - 1))
    def _():
        def outer(i, _):
            si = scores_ref[i]

            def inner(j, r):
                sj = scores_ref[j]
                gt = (sj > si) | ((sj == si) & (j < i))
                return r + gt.astype(jnp.int32)

            r = lax.fori_loop(0, _NT, inner, 0)

            @pl.when(r < _NB)
            def _():
                top_ref[r] = i

            return 0

        lax.fori_loop(0, _NT, outer, 0)


_CC2 = 16
_NCH = _C // _CC2


def _copy_body(idx_ref, img_ref, out_ref, src_bufs, stage, rd_sem, wr_sem):
    g = pl.program_id(0)

    def start_reads(gg, ring):
        c0 = gg * _CC2
        for r in range(_NB):
            t = idx_ref[r]
            sh = pl.multiple_of((t // _NW) * _T, _T)
            swa = pl.multiple_of(((t % _NW) // 2) * 128, 128)
            src = img_ref.at[pl.ds(c0, _CC2), pl.ds(sh, _T), pl.ds(swa, 128)]
            pltpu.make_async_copy(src, src_bufs.at[ring, r], rd_sem.at[ring, r]).start()

    ring = g % 2
    other = (g + 1) % 2

    @pl.when(g == 0)
    def _():
        start_reads(0, 0)

    @pl.when(g + 1 < _NCH)
    def _():
        start_reads(g + 1, other)

    def wait_write(rr, p, gg):
        oh = (p // 3) * _T
        owa = (p % 3) * 128
        dst = out_ref.at[pl.ds(gg * _CC2, _CC2), pl.ds(oh, _T), pl.ds(owa, 128)]
        pltpu.make_async_copy(stage.at[rr, p], dst, wr_sem.at[rr, p]).wait()

    @pl.when(g >= 2)
    def _():
        for p in range(_NB):
            wait_write(ring, p, g - 2)

    for r in range(_NB):
        pltpu.make_async_copy(
            img_ref.at[pl.ds(0, _CC2), pl.ds(0, _T), pl.ds(0, 128)],
            src_bufs.at[ring, r], rd_sem.at[ring, r],
        ).wait()

    for p in range(_NB):
        oh = (p // 3) * _T
        owa = (p % 3) * 128
        kL = (p // 3) * _NW + (p % 3) * 2
        rL, rR = int(_Q[kL]), int(_Q[kL + 1])
        halfL = (idx_ref[rL] % 2) == 1
        halfR = (idx_ref[rR] % 2) == 1
        bufL = src_bufs[ring, rL]
        bufR = src_bufs[ring, rR]
        canonL = jnp.where(halfL, bufL[:, :, _T:], bufL[:, :, :_T])
        canonR = jnp.where(halfR, bufR[:, :, _T:], bufR[:, :, :_T])
        stage[ring, p] = jnp.concatenate([canonL, canonR], axis=-1)
        dst = out_ref.at[pl.ds(g * _CC2, _CC2), pl.ds(oh, _T), pl.ds(owa, 128)]
        pltpu.make_async_copy(stage.at[ring, p], dst, wr_sem.at[ring, p]).start()

    @pl.when(g == _NCH - 1)
    def _():
        for p in range(_NB):
            wait_write(other, p, g - 1)
            wait_write(ring, p, g)


def kernel(img):
    top_idx = pl.pallas_call(
        _score_topk_body,
        grid=(_NH, _NC),
        in_specs=[
            pl.BlockSpec((_CC, _T, _W), lambda th, c: (c, th, 0)),
        ],
        out_specs=pl.BlockSpec(memory_space=pltpu.SMEM),
        out_shape=jax.ShapeDtypeStruct((_NB,), jnp.int32),
        scratch_shapes=[pltpu.SMEM((_NT,), jnp.float32)],
        compiler_params=pltpu.CompilerParams(
            dimension_semantics=("arbitrary", "arbitrary"),
        ),
    )(img)

    out = pl.pallas_call(
        _copy_body,
        grid=(_NCH,),
        in_specs=[
            pl.BlockSpec(memory_space=pltpu.SMEM),
            pl.BlockSpec(memory_space=pl.ANY),
        ],
        out_specs=pl.BlockSpec(memory_space=pl.ANY),
        out_shape=jax.ShapeDtypeStruct((_C, _H, _W), jnp.float32),
        scratch_shapes=[
            pltpu.VMEM((2, _NB, _CC2, _T, 128), jnp.float32),
            pltpu.VMEM((2, _NB, _CC2, _T, 128), jnp.float32),
            pltpu.SemaphoreType.DMA((2, _NB)),
            pltpu.SemaphoreType.DMA((2, _NB)),
        ],
        compiler_params=pltpu.CompilerParams(
            dimension_semantics=("arbitrary",),
        ),
    )(top_idx, img)

    return out
